# baseline (speedup 1.0000x reference)
import functools

import jax
import jax.numpy as jnp
from jax import lax
from jax.experimental import pallas as pl
from jax.experimental.pallas import tpu as pltpu
from jax.experimental.pallas import tpu_sc as plsc

_B = 16384
_D = 64
_NC = 2
_NS = 16
_NW = _NC * _NS
_BPW = _B // _NW
_L = 16
_G = _BPW // _L


@functools.partial(
    pl.kernel,
    mesh=plsc.VectorSubcoreMesh(core_axis_name="c", subcore_axis_name="s"),
    out_type=jax.ShapeDtypeStruct((_B,), jnp.float32),
    compiler_params=pltpu.CompilerParams(
        needs_layout_passes=False, use_tc_tiling_on_sc=False),
    scratch_types=[
        pltpu.VMEM((_BPW,), jnp.int32),
        pltpu.VMEM((_BPW,), jnp.int32),
        pltpu.VMEM((_BPW, _D), jnp.float32),
        pltpu.VMEM((_BPW, _D), jnp.float32),
        pltpu.VMEM((_BPW,), jnp.float32),
        pltpu.VMEM((_BPW,), jnp.float32),
        pltpu.VMEM((_L,), jnp.float32),
        pltpu.VMEM((_BPW,), jnp.float32),
        pltpu.SemaphoreType.DMA,
    ],
)
def _mf_score(uid_hbm, iid_hbm, uemb_hbm, iemb_hbm, ubias_hbm, ibias_hbm,
              gb_hbm, out_hbm,
              uid_v, iid_v, urows_v, irows_v, ub_v, ib_v, gb_v, out_v, sem):
    wid = lax.axis_index("s") * _NC + lax.axis_index("c")
    base = wid * _BPW

    pltpu.sync_copy(uid_hbm.at[pl.ds(base, _BPW)], uid_v)
    pltpu.sync_copy(iid_hbm.at[pl.ds(base, _BPW)], iid_v)
    pltpu.sync_copy(gb_hbm, gb_v)

    c0 = pltpu.async_copy(uemb_hbm.at[uid_v], urows_v, sem)
    c1 = pltpu.async_copy(iemb_hbm.at[iid_v], irows_v, sem)
    c2 = pltpu.async_copy(ubias_hbm.at[uid_v], ub_v, sem)
    c3 = pltpu.async_copy(ibias_hbm.at[iid_v], ib_v, sem)
    c0.wait()
    c1.wait()
    c2.wait()
    c3.wait()

    gb = gb_v[pl.ds(0, _L)]
    lane = lax.iota(jnp.int32, _L)

    def group(g, carry):
        rows = g * _L + lane
        acc = ub_v[pl.ds(g * _L, _L)] + ib_v[pl.ds(g * _L, _L)] + gb
        for d in range(_D):
            col = jnp.full((_L,), d, jnp.int32)
            u = plsc.load_gather(urows_v, [rows, col])
            it = plsc.load_gather(irows_v, [rows, col])
            acc = acc + u * it
        out_v[pl.ds(g * _L, _L)] = acc
        return carry

    lax.fori_loop(0, _G, group, 0)

    pltpu.sync_copy(out_v, out_hbm.at[pl.ds(base, _BPW)])


def kernel(user_ids, item_ids, user_emb, item_emb, user_bias, item_bias,
           global_bias):
    uid = user_ids.astype(jnp.int32)
    iid = item_ids.astype(jnp.int32)
    ub = user_bias.reshape(-1)
    ib = item_bias.reshape(-1)
    gb = jnp.broadcast_to(global_bias.reshape(-1)[:1], (_L,))
    return _mf_score(uid, iid, user_emb, item_emb, ub, ib, gb)
